# baseline (speedup 1.0000x reference)
import jax
import jax.numpy as jnp
from jax import lax
from jax.experimental import pallas as pl
from jax.experimental.pallas import tpu as pltpu
from jax.experimental.pallas import tpu_sc as plsc

_D = 128
_HD = 64
_CH = 128
_SUP = 4
_NSUB = 16
_WIN = 15360
_N1, _N2, _N3 = 30000, 16384, 4096


def _sc_segsum(n_pad, e_pad, supers, lo, window):
    mesh = plsc.VectorSubcoreMesh(core_axis_name="c", subcore_axis_name="s")
    chunks_per_sub = e_pad // _CH // _NSUB
    stripe = n_pad // _NSUB
    nz = stripe // _CH

    def body(src1, dst1, xb, agg_lo, agg_hi, cnt0_out, cnt1_out,
             sb0, sb1, sb2, sb3, db0, db1, db2, db3,
             rows, ones2, acc, cnta, *sems):
        sbufs = [sb0, sb1, sb2, sb3]
        dbufs = [db0, db1, db2, db3]
        cid = lax.axis_index("c")
        sid = lax.axis_index("s")

        z16 = jnp.zeros((16,), jnp.float32)
        o16 = jnp.ones((16,), jnp.float32)

        def zone(r, c2):
            ones2[r, pl.ds(0, 16)] = z16
            return c2
        lax.fori_loop(0, _CH, zone, 0)

        def zrow(r, c2):
            for j in range(_HD // 16):
                rows[0, r, pl.ds(j * 16, 16)] = z16
            return c2
        lax.fori_loop(0, _CH, zrow, 0)

        r0 = sid * stripe
        for k in range(nz):
            pltpu.sync_copy(rows.at[0], acc.at[pl.ds(r0 + k * _CH, _CH)])
            pltpu.sync_copy(ones2, cnta.at[pl.ds(r0 + k * _CH, _CH)])

        def fone(r, c2):
            ones2[r, pl.ds(0, 16)] = o16
            return c2
        lax.fori_loop(0, _CH, fone, 0)
        plsc.subcore_barrier()

        def super_body(g, carry):
            e0 = (sid * chunks_per_sub + g * _SUP) * _CH
            for i in range(_SUP):
                pltpu.sync_copy(src1.at[pl.ds(e0 + i * _CH, _CH)], sbufs[i])
                pltpu.sync_copy(dst1.at[pl.ds(e0 + i * _CH, _CH)], dbufs[i])
            dumm = (0 if window is None else window) + sid * 64 + \
                lax.iota(jnp.int32, 16)
            for i in range(_SUP):
                for j in range(_CH // 16):
                    v = sbufs[i][pl.ds(j * 16, 16)]
                    sbufs[i][pl.ds(j * 16, 16)] = v * 2 + cid
                    if window is not None:
                        d = dbufs[i][pl.ds(j * 16, 16)] - lo
                        ok = (d >= 0) & (d < window)
                        dbufs[i][pl.ds(j * 16, 16)] = jnp.where(ok, d, dumm)
            descs = [pltpu.async_copy(xb.at[sbufs[i]], rows.at[i], sems[i])
                     for i in range(_SUP)]
            for i in range(_SUP):
                @pl.when(cid == (i % 2))
                def _(i=i):
                    pltpu.sync_copy(ones2, cnta.at[dbufs[i]], add=True)
                descs[i].wait()
                pltpu.sync_copy(rows.at[i], acc.at[dbufs[i]], add=True)
            return carry
        lax.fori_loop(0, supers, super_body, 0)
        plsc.subcore_barrier()

        for k in range(nz):
            pltpu.sync_copy(acc.at[pl.ds(r0 + k * _CH, _CH)], rows.at[0])

            @pl.when(cid == 0)
            def _(k=k):
                pltpu.sync_copy(rows.at[0],
                                agg_lo.at[pl.ds(r0 + k * _CH, _CH)])

            @pl.when(cid == 1)
            def _(k=k):
                pltpu.sync_copy(rows.at[0],
                                agg_hi.at[pl.ds(r0 + k * _CH, _CH)])

        for k in range(nz):
            pltpu.sync_copy(cnta.at[pl.ds(r0 + k * _CH, _CH)], ones2)

            @pl.when(cid == 0)
            def _(k=k):
                pltpu.sync_copy(ones2,
                                cnt0_out.at[pl.ds(r0 + k * _CH, _CH)])

            @pl.when(cid == 1)
            def _(k=k):
                pltpu.sync_copy(ones2,
                                cnt1_out.at[pl.ds(r0 + k * _CH, _CH)])

    scratch = [
        pltpu.VMEM((_CH,), jnp.int32),
        pltpu.VMEM((_CH,), jnp.int32),
        pltpu.VMEM((_CH,), jnp.int32),
        pltpu.VMEM((_CH,), jnp.int32),
        pltpu.VMEM((_CH,), jnp.int32),
        pltpu.VMEM((_CH,), jnp.int32),
        pltpu.VMEM((_CH,), jnp.int32),
        pltpu.VMEM((_CH,), jnp.int32),
        pltpu.VMEM((_SUP, _CH, _HD), jnp.float32),
        pltpu.VMEM((_CH, 16), jnp.float32),
        pltpu.VMEM_SHARED((n_pad, _HD), jnp.float32),
        pltpu.VMEM_SHARED((n_pad, 16), jnp.float32),
    ] + [pltpu.SemaphoreType.DMA] * _SUP

    return pl.kernel(
        body,
        out_type=[
            jax.ShapeDtypeStruct((n_pad, _HD), jnp.float32),
            jax.ShapeDtypeStruct((n_pad, _HD), jnp.float32),
            jax.ShapeDtypeStruct((n_pad, 16), jnp.float32),
            jax.ShapeDtypeStruct((n_pad, 16), jnp.float32),
        ],
        mesh=mesh,
        scratch_types=scratch,
        compiler_params=pltpu.CompilerParams(use_tc_tiling_on_sc=False),
    )


def _dense(n_dst, b):
    def body(alo, ahi, cnts, cnts1, x, wlt, wrt, bias, alpha, out):
        cnt = cnts[...][:, 0:1] + cnts1[...][:, 0:1]
        r = 1.0 / jnp.maximum(cnt, 1.0)
        z = jnp.concatenate([alo[...], ahi[...]], axis=1) * r
        h = jnp.dot(z, wlt[...], preferred_element_type=jnp.float32)
        h = h + jnp.dot(x[...], wrt[...], preferred_element_type=jnp.float32)
        h = h + bias[...]
        out[...] = jnp.where(h > 0.0, h, alpha[...] * h)

    return pl.pallas_call(
        body,
        grid=(n_dst // b,),
        in_specs=[
            pl.BlockSpec((b, _HD), lambda i: (i, 0)),
            pl.BlockSpec((b, _HD), lambda i: (i, 0)),
            pl.BlockSpec((b, _NSUB), lambda i: (i, 0)),
            pl.BlockSpec((b, _NSUB), lambda i: (i, 0)),
            pl.BlockSpec((b, _D), lambda i: (i, 0)),
            pl.BlockSpec((_D, _D), lambda i: (0, 0)),
            pl.BlockSpec((_D, _D), lambda i: (0, 0)),
            pl.BlockSpec((1, _D), lambda i: (0, 0)),
            pl.BlockSpec((1, _D), lambda i: (0, 0)),
        ],
        out_specs=pl.BlockSpec((b, _D), lambda i: (i, 0)),
        out_shape=jax.ShapeDtypeStruct((n_dst, _D), jnp.float32),
    )


def _round_up(v, m):
    return (v + m - 1) // m * m


def _layer(x, e, n_dst, Wl, bl, Wr, alpha, b):
    n_src = x.shape[0]
    n_edge = e.shape[1]
    e_pad = _round_up(n_edge, _NSUB * _SUP * _CH)
    supers = e_pad // (_NSUB * _SUP * _CH)

    src = e[0].astype(jnp.int32)
    dst = e[1].astype(jnp.int32)
    if e_pad > n_edge:
        pad = e_pad - n_edge
        src = jnp.concatenate([src, jnp.zeros((pad,), jnp.int32)])
        dst = jnp.concatenate([dst, jnp.full((pad,), n_dst, jnp.int32)])
    xb = x.reshape(2 * n_src, _HD)

    if n_dst + (1 if e_pad > n_edge else 0) <= 16384:
        n_pad = _round_up(n_dst, _NSUB * _CH)
        alo, ahi, cnts, cnts1 = _sc_segsum(n_pad, e_pad, supers, 0, None)(
            src, dst, xb)
        alo, ahi = alo[:n_dst], ahi[:n_dst]
        cnts, cnts1 = cnts[:n_dst], cnts1[:n_dst]
    else:
        n_pad = _WIN + 1024
        alos, ahis, cntss, cntss1 = [], [], [], []
        for w0 in range(0, n_dst, _WIN):
            take = min(_WIN, n_dst - w0)
            a0, a1, c, c1 = _sc_segsum(n_pad, e_pad, supers, w0, _WIN)(
                src, dst, xb)
            alos.append(a0[:take])
            ahis.append(a1[:take])
            cntss.append(c[:take])
            cntss1.append(c1[:take])
        alo = jnp.concatenate(alos)
        ahi = jnp.concatenate(ahis)
        cnts = jnp.concatenate(cntss)
        cnts1 = jnp.concatenate(cntss1)

    h = _dense(n_dst, b)(
        alo, ahi, cnts, cnts1, x,
        Wl.T, Wr.T, bl.reshape(1, _D), alpha.reshape(1, _D))
    return h


def kernel(x, edge_index1, edge_index2, edge_index3,
           Wl1, bl1, Wr1, alpha1,
           Wl2, bl2, Wr2, alpha2,
           Wl3, bl3, Wr3, alpha3):
    h = _layer(x, edge_index1, _N1, Wl1, bl1, Wr1, alpha1, 2000)
    h = _layer(h, edge_index2, _N2, Wl2, bl2, Wr2, alpha2, 2048)
    h = _layer(h, edge_index3, _N3, Wl3, bl3, Wr3, alpha3, 2048)
    return h


# speedup vs baseline: 1.0509x; 1.0509x over previous
import jax
import jax.numpy as jnp
from jax import lax
from jax.experimental import pallas as pl
from jax.experimental.pallas import tpu as pltpu
from jax.experimental.pallas import tpu_sc as plsc

_D = 128
_HD = 64
_CH = 128
_SUP = 4
_NSUB = 16
_WIN = 15360
_N1, _N2, _N3 = 30000, 16384, 4096


def _sc_segsum(n_pad, e_pad, supers, lo, window):
    mesh = plsc.VectorSubcoreMesh(core_axis_name="c", subcore_axis_name="s")
    chunks_per_sub = e_pad // _CH // _NSUB
    stripe = n_pad // _NSUB
    nz = stripe // _CH

    def body(src1, dst1, xb, agg_lo, agg_hi, cnt_out,
             sb0, sb1, sb2, sb3, db0, db1, db2, db3,
             rows, ones2, acc, cnta, *sems):
        sbufs = [sb0, sb1, sb2, sb3]
        dbufs = [db0, db1, db2, db3]
        cid = lax.axis_index("c")
        sid = lax.axis_index("s")

        z16 = jnp.zeros((16,), jnp.float32)
        o16 = jnp.ones((16,), jnp.float32)

        def zone(r, c2):
            ones2[r, pl.ds(0, 16)] = z16
            return c2
        lax.fori_loop(0, _CH, zone, 0)

        def zrow(r, c2):
            for j in range(_HD // 16):
                rows[0, r, pl.ds(j * 16, 16)] = z16
            return c2
        lax.fori_loop(0, _CH, zrow, 0)

        r0 = sid * stripe
        for k in range(nz):
            pltpu.sync_copy(rows.at[0], acc.at[pl.ds(r0 + k * _CH, _CH)])
            pltpu.sync_copy(ones2, cnta.at[pl.ds(r0 + k * _CH, _CH)])

        def fone(r, c2):
            ones2[r, pl.ds(0, 16)] = o16
            return c2
        lax.fori_loop(0, _CH, fone, 0)
        plsc.subcore_barrier()

        def super_body(g, carry):
            e0 = (sid * chunks_per_sub + g * _SUP) * _CH
            for i in range(_SUP):
                pltpu.sync_copy(src1.at[pl.ds(e0 + i * _CH, _CH)], sbufs[i])
                pltpu.sync_copy(dst1.at[pl.ds(e0 + i * _CH, _CH)], dbufs[i])
            dumm = (0 if window is None else window) + sid * 64 + \
                lax.iota(jnp.int32, 16)
            for i in range(_SUP):
                for j in range(_CH // 16):
                    v = sbufs[i][pl.ds(j * 16, 16)]
                    sbufs[i][pl.ds(j * 16, 16)] = v * 2 + cid
                    if window is not None:
                        d = dbufs[i][pl.ds(j * 16, 16)] - lo
                        ok = (d >= 0) & (d < window)
                        dbufs[i][pl.ds(j * 16, 16)] = jnp.where(ok, d, dumm)
            descs = [pltpu.async_copy(xb.at[sbufs[i]], rows.at[i], sems[i])
                     for i in range(_SUP)]
            for i in range(_SUP):
                @pl.when(cid == 0)
                def _(i=i):
                    pltpu.sync_copy(ones2, cnta.at[dbufs[i]], add=True)
            for i in range(_SUP):
                descs[i].wait()
                pltpu.sync_copy(rows.at[i], acc.at[dbufs[i]], add=True)
            return carry
        lax.fori_loop(0, supers, super_body, 0)
        plsc.subcore_barrier()

        for k in range(nz):
            pltpu.sync_copy(acc.at[pl.ds(r0 + k * _CH, _CH)], rows.at[0])

            @pl.when(cid == 0)
            def _(k=k):
                pltpu.sync_copy(rows.at[0],
                                agg_lo.at[pl.ds(r0 + k * _CH, _CH)])

            @pl.when(cid == 1)
            def _(k=k):
                pltpu.sync_copy(rows.at[0],
                                agg_hi.at[pl.ds(r0 + k * _CH, _CH)])

        for k in range(nz):
            pltpu.sync_copy(cnta.at[pl.ds(r0 + k * _CH, _CH)], ones2)

            @pl.when(cid == 0)
            def _(k=k):
                pltpu.sync_copy(ones2,
                                cnt_out.at[pl.ds(r0 + k * _CH, _CH)])

    scratch = [
        pltpu.VMEM((_CH,), jnp.int32),
        pltpu.VMEM((_CH,), jnp.int32),
        pltpu.VMEM((_CH,), jnp.int32),
        pltpu.VMEM((_CH,), jnp.int32),
        pltpu.VMEM((_CH,), jnp.int32),
        pltpu.VMEM((_CH,), jnp.int32),
        pltpu.VMEM((_CH,), jnp.int32),
        pltpu.VMEM((_CH,), jnp.int32),
        pltpu.VMEM((_SUP, _CH, _HD), jnp.float32),
        pltpu.VMEM((_CH, 16), jnp.float32),
        pltpu.VMEM_SHARED((n_pad, _HD), jnp.float32),
        pltpu.VMEM_SHARED((n_pad, 16), jnp.float32),
    ] + [pltpu.SemaphoreType.DMA] * _SUP

    return pl.kernel(
        body,
        out_type=[
            jax.ShapeDtypeStruct((n_pad, _HD), jnp.float32),
            jax.ShapeDtypeStruct((n_pad, _HD), jnp.float32),
            jax.ShapeDtypeStruct((n_pad, 16), jnp.float32),
        ],
        mesh=mesh,
        scratch_types=scratch,
        compiler_params=pltpu.CompilerParams(use_tc_tiling_on_sc=False),
    )


def _dense(n_dst, b):
    def body(alo, ahi, cnts, x, wlt, wrt, bias, alpha, out):
        cnt = cnts[...][:, 0:1]
        r = 1.0 / jnp.maximum(cnt, 1.0)
        z = jnp.concatenate([alo[...], ahi[...]], axis=1) * r
        h = jnp.dot(z, wlt[...], preferred_element_type=jnp.float32)
        h = h + jnp.dot(x[...], wrt[...], preferred_element_type=jnp.float32)
        h = h + bias[...]
        out[...] = jnp.where(h > 0.0, h, alpha[...] * h)

    return pl.pallas_call(
        body,
        grid=(n_dst // b,),
        in_specs=[
            pl.BlockSpec((b, _HD), lambda i: (i, 0)),
            pl.BlockSpec((b, _HD), lambda i: (i, 0)),
            pl.BlockSpec((b, _NSUB), lambda i: (i, 0)),
            pl.BlockSpec((b, _D), lambda i: (i, 0)),
            pl.BlockSpec((_D, _D), lambda i: (0, 0)),
            pl.BlockSpec((_D, _D), lambda i: (0, 0)),
            pl.BlockSpec((1, _D), lambda i: (0, 0)),
            pl.BlockSpec((1, _D), lambda i: (0, 0)),
        ],
        out_specs=pl.BlockSpec((b, _D), lambda i: (i, 0)),
        out_shape=jax.ShapeDtypeStruct((n_dst, _D), jnp.float32),
    )


def _round_up(v, m):
    return (v + m - 1) // m * m


def _layer(x, e, n_dst, Wl, bl, Wr, alpha, b):
    n_src = x.shape[0]
    n_edge = e.shape[1]
    e_pad = _round_up(n_edge, _NSUB * _SUP * _CH)
    supers = e_pad // (_NSUB * _SUP * _CH)

    src = e[0].astype(jnp.int32)
    dst = e[1].astype(jnp.int32)
    if e_pad > n_edge:
        pad = e_pad - n_edge
        src = jnp.concatenate([src, jnp.zeros((pad,), jnp.int32)])
        dst = jnp.concatenate([dst, jnp.full((pad,), n_dst, jnp.int32)])
    xb = x.reshape(2 * n_src, _HD)

    if n_dst + (1 if e_pad > n_edge else 0) <= 16384:
        n_pad = _round_up(n_dst, _NSUB * _CH)
        alo, ahi, cnts = _sc_segsum(n_pad, e_pad, supers, 0, None)(
            src, dst, xb)
        alo, ahi, cnts = alo[:n_dst], ahi[:n_dst], cnts[:n_dst]
    else:
        n_pad = _WIN + 1024
        alos, ahis, cntss = [], [], []
        for w0 in range(0, n_dst, _WIN):
            take = min(_WIN, n_dst - w0)
            a0, a1, c = _sc_segsum(n_pad, e_pad, supers, w0, _WIN)(
                src, dst, xb)
            alos.append(a0[:take])
            ahis.append(a1[:take])
            cntss.append(c[:take])
        alo = jnp.concatenate(alos)
        ahi = jnp.concatenate(ahis)
        cnts = jnp.concatenate(cntss)

    h = _dense(n_dst, b)(
        alo, ahi, cnts, x,
        Wl.T, Wr.T, bl.reshape(1, _D), alpha.reshape(1, _D))
    return h


def kernel(x, edge_index1, edge_index2, edge_index3,
           Wl1, bl1, Wr1, alpha1,
           Wl2, bl2, Wr2, alpha2,
           Wl3, bl3, Wr3, alpha3):
    h = _layer(x, edge_index1, _N1, Wl1, bl1, Wr1, alpha1, 2000)
    h = _layer(h, edge_index2, _N2, Wl2, bl2, Wr2, alpha2, 2048)
    h = _layer(h, edge_index3, _N3, Wl3, bl3, Wr3, alpha3, 2048)
    return h
